# baseline (device time: 109671 ns/iter reference)
import jax
import jax.numpy as jnp
from jax import lax
from jax.experimental import pallas as pl
from jax.experimental.pallas import tpu as pltpu

N_DEV = 8
N_TOK = 2048
D_MODEL = 512
D_FF = 1024
N_EXP = 32
E_LOCAL = 4
CHUNK = N_TOK // N_DEV


def kernel(x, router_W, route_idx, expert_W):
    def body(x_ref, rw_ref, idx_ref, ew_ref, out_ref,
             comm_ref, gates_ref, send_sems, recv_sems):
        my = lax.axis_index("i")
        left = lax.rem(my - 1 + N_DEV, N_DEV)
        right = lax.rem(my + 1, N_DEV)

        scores = jnp.dot(x_ref[:, :], rw_ref[:, :],
                         preferred_element_type=jnp.float32)
        m = jnp.max(scores, axis=1, keepdims=True)
        p = jnp.exp(scores - m)
        probs = p / jnp.sum(p, axis=1, keepdims=True)
        e_iota = lax.broadcasted_iota(jnp.int32, (N_TOK, N_EXP), 1)
        idx0 = idx_ref[:, 0:1]
        idx1 = idx_ref[:, 1:2]
        g0 = jnp.sum(jnp.where(e_iota == idx0, probs, 0.0), axis=1,
                     keepdims=True)
        g1 = jnp.sum(jnp.where(e_iota == idx1, probs, 0.0), axis=1,
                     keepdims=True)
        gs = g0 + g1
        gates_ref[:, 0:1] = g0 / gs
        gates_ref[:, 1:2] = g1 / gs

        def chunk_partial(c):
            rows = pl.ds(c * CHUNK, CHUNK)
            xr = x_ref[rows, :]
            xi0 = idx_ref[rows, 0:1]
            xi1 = idx_ref[rows, 1:2]
            w0 = gates_ref[rows, 0:1]
            w1 = gates_ref[rows, 1:2]
            acc = jnp.zeros((CHUNK, D_FF), jnp.float32)
            for j in range(E_LOCAL):
                ge = my * E_LOCAL + j
                coeff = (jnp.where(xi0 == ge, w0, 0.0)
                         + jnp.where(xi1 == ge, w1, 0.0))
                acc = acc + jnp.dot(xr * coeff, ew_ref[j],
                                    preferred_element_type=jnp.float32)
            return acc

        barrier = pltpu.get_barrier_semaphore()
        for nbr in (left, right):
            pl.semaphore_signal(barrier, inc=1, device_id=(nbr,),
                                device_id_type=pl.DeviceIdType.MESH)
        pl.semaphore_wait(barrier, 2)

        comm_ref[0, :, :] = chunk_partial(left)

        for s in range(N_DEV - 1):
            rdma = pltpu.make_async_remote_copy(
                src_ref=comm_ref.at[s],
                dst_ref=comm_ref.at[s + 1],
                send_sem=send_sems.at[s],
                recv_sem=recv_sems.at[s],
                device_id=(right,),
                device_id_type=pl.DeviceIdType.MESH,
            )
            rdma.start()
            c = lax.rem(my - 2 - s + 2 * N_DEV, N_DEV)
            part = chunk_partial(c)
            rdma.wait()
            if s < N_DEV - 2:
                comm_ref[s + 1, :, :] = comm_ref[s + 1, :, :] + part
            else:
                out_ref[:, :] = comm_ref[s + 1, :, :] + part

    return pl.pallas_call(
        body,
        out_shape=jax.ShapeDtypeStruct((CHUNK, D_FF), jnp.float32),
        in_specs=[
            pl.BlockSpec(memory_space=pltpu.VMEM),
            pl.BlockSpec(memory_space=pltpu.VMEM),
            pl.BlockSpec(memory_space=pltpu.VMEM),
            pl.BlockSpec(memory_space=pltpu.VMEM),
        ],
        out_specs=pl.BlockSpec(memory_space=pltpu.VMEM),
        scratch_shapes=[
            pltpu.VMEM((N_DEV, CHUNK, D_FF), jnp.float32),
            pltpu.VMEM((N_TOK, 2), jnp.float32),
            pltpu.SemaphoreType.DMA((N_DEV - 1,)),
            pltpu.SemaphoreType.DMA((N_DEV - 1,)),
        ],
        compiler_params=pltpu.CompilerParams(collective_id=0),
    )(x, router_W, route_idx, expert_W)
